# Initial kernel scaffold; baseline (speedup 1.0000x reference)
import jax
import jax.numpy as jnp
from jax.experimental import pallas as pl


def kernel(z, q, codebook, Wq, bq, Wk, bk, Wv, bv, Wp, bp, gq, gk):
    raise NotImplementedError("write your pallas kernel here")


# speedup vs baseline: 1.1134x; 1.1134x over previous
import functools
import math

import jax
import jax.numpy as jnp
from jax import lax
from jax.experimental import pallas as pl
from jax.experimental.pallas import tpu as pltpu

B, C, T, V, Q, H = 4, 64, 1024, 8192, 256, 8
DH = C // H
VB = 512
NV = V // VB
P_POOL = B * Q
P_FULL = B * T
P = P_POOL + P_FULL
EPS = 1e-5


def _rms_rows(x, g_tile):
    pieces = []
    for h in range(H):
        xh = x[:, h * DH:(h + 1) * DH]
        ss = jnp.sum(xh * xh, axis=1, keepdims=True)
        inv = lax.rsqrt(ss * (1.0 / DH) + EPS)
        pieces.append(jnp.broadcast_to(inv, xh.shape))
    return x * jnp.concatenate(pieces, axis=1) * g_tile


def _expand_heads(c):
    return jnp.concatenate(
        [jnp.broadcast_to(c[:, h:h + 1], (c.shape[0], DH)) for h in range(H)],
        axis=1)


def _interp_matrix():
    t = lax.broadcasted_iota(jnp.int32, (Q, T), 1).astype(jnp.float32)
    s = lax.broadcasted_iota(jnp.int32, (Q, T), 0).astype(jnp.float32)
    src = jnp.clip((t + 0.5) * (Q / T) - 0.5, 0.0, Q - 1.0)
    i0 = jnp.floor(src)
    w = src - i0
    i1 = jnp.minimum(i0 + 1.0, Q - 1.0)
    return (s == i0) * (1.0 - w) + (s == i1) * w


def _fused_kernel(z_btc, cb_blk, WqT, bq, WkT, bk, WvT, bv, WpT, bp, gq_t,
                  gk_t, zhat_out, perp_out, wq_s, best_s, bidx_s, zqT_s,
                  plog_s):
    i = pl.program_id(0)

    @pl.when(i == 0)
    def _prep_queries():
        z4 = z_btc[...].reshape(B, Q, T // Q, C)
        pooled = ((z4[:, :, 0, :] + z4[:, :, 1, :]) + z4[:, :, 2, :]
                  + z4[:, :, 3, :]) * (Q / T)
        hs = jnp.concatenate(
            [pooled.reshape(P_POOL, C), z_btc[...].reshape(P_FULL, C)], axis=0)
        qv = jnp.dot(hs, WqT[...], preferred_element_type=jnp.float32) + bq[...]
        qvr = _rms_rows(qv, gq_t[...])
        c = jnp.dot(hs, WpT[...], preferred_element_type=jnp.float32) + bp[...]
        wq_s[...] = _expand_heads(c) * qvr

    @pl.when(i < NV)
    def _logits_phase():
        kk = jnp.dot(cb_blk[...], WkT[...],
                     preferred_element_type=jnp.float32) + bk[...]
        kvf = _rms_rows(kk, gk_t[...])
        raw = lax.dot_general(wq_s[...], kvf, (((1,), (1,)), ((), ())),
                              preferred_element_type=jnp.float32)
        logits = raw * (1.0 / math.sqrt(DH)) / math.sqrt(H)
        tmax = jnp.max(logits, axis=1, keepdims=True)
        targ = jnp.argmax(logits, axis=1).astype(jnp.int32).reshape(P, 1) \
            + i * VB

        @pl.when(i == 0)
        def _():
            best_s[...] = tmax
            bidx_s[...] = targ

        @pl.when(i > 0)
        def _():
            upd = tmax > best_s[...]
            best_s[...] = jnp.where(upd, tmax, best_s[...])
            bidx_s[...] = jnp.where(upd, targ, bidx_s[...])

    @pl.when(i >= NV)
    def _assign_phase():
        j = i - NV
        value = jnp.dot(cb_blk[...], WvT[...],
                        preferred_element_type=jnp.float32) + bv[...]
        lane = lax.broadcasted_iota(jnp.int32, (1, VB), 1) + j * VB
        onehot = (bidx_s[0:P_POOL, :] == lane).astype(jnp.float32)
        zq_part = lax.dot_general(value, onehot, (((0,), (1,)), ((), ())),
                                  preferred_element_type=jnp.float32)
        cmp = (bidx_s[P_POOL:P, :] == lane).astype(jnp.float32)
        counts = jnp.sum(cmp, axis=0, keepdims=True)
        p = counts * (1.0 / P_FULL)
        part = jnp.sum(p * jnp.log(p + 1e-7), axis=(0, 1), keepdims=True)

        @pl.when(j == 0)
        def _():
            zqT_s[...] = zq_part
            plog_s[...] = part

        @pl.when(j > 0)
        def _():
            zqT_s[...] = zqT_s[...] + zq_part
            plog_s[...] = plog_s[...] + part

        @pl.when(j == NV - 1)
        def _finalize():
            M = _interp_matrix()
            for b in range(B):
                zhat_out[b, :, :] = jnp.dot(zqT_s[:, b * Q:(b + 1) * Q], M,
                                            preferred_element_type=jnp.float32)
            perp_out[...] = jnp.exp(-plog_s[...])


@functools.partial(jax.jit, static_argnames=())
def _run(z, codebook, Wq, bq, Wk, bk, Wv, bv, Wp, bp, gq, gk):
    z_btc = jnp.transpose(z, (0, 2, 1))
    zhat, perp = pl.pallas_call(
        _fused_kernel,
        grid=(2 * NV,),
        in_specs=[
            pl.BlockSpec((B, T, C), lambda i: (0, 0, 0)),
            pl.BlockSpec((VB, C), lambda i: (i % NV, 0)),
            pl.BlockSpec((C, C), lambda i: (0, 0)),
            pl.BlockSpec((1, C), lambda i: (0, 0)),
            pl.BlockSpec((C, C), lambda i: (0, 0)),
            pl.BlockSpec((1, C), lambda i: (0, 0)),
            pl.BlockSpec((C, C), lambda i: (0, 0)),
            pl.BlockSpec((1, C), lambda i: (0, 0)),
            pl.BlockSpec((C, H), lambda i: (0, 0)),
            pl.BlockSpec((1, H), lambda i: (0, 0)),
            pl.BlockSpec((1, C), lambda i: (0, 0)),
            pl.BlockSpec((1, C), lambda i: (0, 0)),
        ],
        out_specs=[
            pl.BlockSpec((B, C, T), lambda i: (0, 0, 0)),
            pl.BlockSpec((1, 1), lambda i: (0, 0)),
        ],
        out_shape=[
            jax.ShapeDtypeStruct((B, C, T), jnp.float32),
            jax.ShapeDtypeStruct((1, 1), jnp.float32),
        ],
        scratch_shapes=[
            pltpu.VMEM((P, C), jnp.float32),
            pltpu.VMEM((P, 1), jnp.float32),
            pltpu.VMEM((P, 1), jnp.int32),
            pltpu.VMEM((C, P_POOL), jnp.float32),
            pltpu.VMEM((1, 1), jnp.float32),
        ],
    )(z_btc, codebook, Wq.T, bq.reshape(1, C), Wk.T, bk.reshape(1, C), Wv.T,
      bv.reshape(1, C), Wp.T, bp.reshape(1, H), jnp.tile(gq, H).reshape(1, C),
      jnp.tile(gk, H).reshape(1, C))
    return zhat, perp[0, 0]


def kernel(z, q, codebook, Wq, bq, Wk, bk, Wv, bv, Wp, bp, gq, gk):
    del q
    return _run(z, codebook, Wq, bq, Wk, bk, Wv, bv, Wp, bp, gq, gk)


# speedup vs baseline: 1.3981x; 1.2557x over previous
import functools
import math

import jax
import jax.numpy as jnp
from jax import lax
from jax.experimental import pallas as pl
from jax.experimental.pallas import tpu as pltpu

B, C, T, V, Q, H = 4, 64, 1024, 8192, 256, 8
DH = C // H
VB = 512
NV = V // VB
P_POOL = B * Q
P_FULL = B * T
P = P_POOL + P_FULL
EPS = 1e-5
IBIG = 2**31 - 1


def _rms_rows(x, g_tile):
    pieces = []
    for h in range(H):
        xh = x[:, h * DH:(h + 1) * DH]
        ss = jnp.sum(xh * xh, axis=1, keepdims=True)
        inv = lax.rsqrt(ss * (1.0 / DH) + EPS)
        pieces.append(jnp.broadcast_to(inv, xh.shape))
    return x * jnp.concatenate(pieces, axis=1) * g_tile


def _rms_cols(x, g_col):
    pieces = []
    for h in range(H):
        xh = x[h * DH:(h + 1) * DH, :]
        ss = jnp.sum(xh * xh, axis=0, keepdims=True)
        inv = lax.rsqrt(ss * (1.0 / DH) + EPS)
        pieces.append(jnp.broadcast_to(inv, xh.shape))
    return x * jnp.concatenate(pieces, axis=0) * g_col


def _expand_head_rows(c):
    return jnp.concatenate(
        [jnp.broadcast_to(c[h:h + 1, :], (DH, c.shape[1])) for h in range(H)],
        axis=0)


def _interp_matrix():
    t = lax.broadcasted_iota(jnp.int32, (Q, T), 1).astype(jnp.float32)
    s = lax.broadcasted_iota(jnp.int32, (Q, T), 0).astype(jnp.float32)
    src = jnp.clip((t + 0.5) * (Q / T) - 0.5, 0.0, Q - 1.0)
    i0 = jnp.floor(src)
    w = src - i0
    i1 = jnp.minimum(i0 + 1.0, Q - 1.0)
    return (s == i0) * (1.0 - w) + (s == i1) * w


def _fused_kernel(z_btc, cb_blk, WqT, bq, WkT, bk, WvT, bv, WpT, bp, gq_col,
                  gk_t, zhat_out, perp_out, wqT_s, best_s, bidxr_s, bidxc_s,
                  zqT_s, plog_s):
    i = pl.program_id(0)

    @pl.when(i == 0)
    def _prep_queries():
        z4 = z_btc[...].reshape(B, Q, T // Q, C)
        pooled = ((z4[:, :, 0, :] + z4[:, :, 1, :]) + z4[:, :, 2, :]
                  + z4[:, :, 3, :]) * (Q / T)
        hs = jnp.concatenate(
            [pooled.reshape(P_POOL, C), z_btc[...].reshape(P_FULL, C)], axis=0)
        qv = jnp.dot(hs, WqT[...], preferred_element_type=jnp.float32) + bq[...]
        c = jnp.dot(hs, WpT[...], preferred_element_type=jnp.float32) + bp[...]
        qvrT = _rms_cols(jnp.transpose(qv, (1, 0)), gq_col[...])
        wqT_s[...] = _expand_head_rows(jnp.transpose(c, (1, 0))) * qvrT

    @pl.when(i < NV)
    def _logits_phase():
        kk = jnp.dot(cb_blk[...], WkT[...],
                     preferred_element_type=jnp.float32) + bk[...]
        kvf = _rms_rows(kk, gk_t[...])
        logits = lax.dot_general(kvf, wqT_s[...], (((1,), (0,)), ((), ())),
                                 preferred_element_type=jnp.float32)
        tmax = jnp.max(logits, axis=0, keepdims=True)
        riota = lax.broadcasted_iota(jnp.int32, (VB, P), 0)
        cand = jnp.where(logits == jnp.broadcast_to(tmax, (VB, P)), riota,
                         IBIG)
        targ = jnp.min(cand, axis=0, keepdims=True) + i * VB

        @pl.when(i == 0)
        def _():
            best_s[...] = tmax
            bidxr_s[...] = targ

        @pl.when(i > 0)
        def _():
            upd = tmax > best_s[...]
            best_s[...] = jnp.where(upd, tmax, best_s[...])
            bidxr_s[...] = jnp.where(upd, targ, bidxr_s[...])

    @pl.when(i >= NV)
    def _assign_phase():
        j = i - NV

        @pl.when(i == NV)
        def _():
            bidxc_s[...] = jnp.transpose(bidxr_s[...], (1, 0))

        value = jnp.dot(cb_blk[...], WvT[...],
                        preferred_element_type=jnp.float32) + bv[...]
        lane = lax.broadcasted_iota(jnp.int32, (1, VB), 1) + j * VB
        onehot = (bidxc_s[0:P_POOL, :] == lane).astype(jnp.float32)
        zq_part = lax.dot_general(value, onehot, (((0,), (1,)), ((), ())),
                                  preferred_element_type=jnp.float32)
        cmp = (bidxc_s[P_POOL:P, :] == lane).astype(jnp.float32)
        counts = jnp.sum(cmp, axis=0, keepdims=True)
        p = counts * (1.0 / P_FULL)
        part = jnp.sum(p * jnp.log(p + 1e-7), axis=(0, 1), keepdims=True)

        @pl.when(j == 0)
        def _():
            zqT_s[...] = zq_part
            plog_s[...] = part

        @pl.when(j > 0)
        def _():
            zqT_s[...] = zqT_s[...] + zq_part
            plog_s[...] = plog_s[...] + part

        @pl.when(j == NV - 1)
        def _finalize():
            M = _interp_matrix()
            for b in range(B):
                zhat_out[b, :, :] = jnp.dot(zqT_s[:, b * Q:(b + 1) * Q], M,
                                            preferred_element_type=jnp.float32)
            perp_out[...] = jnp.exp(-plog_s[...])


@functools.partial(jax.jit, static_argnames=())
def _run(z, codebook, Wq, bq, Wk, bk, Wv, bv, Wp, bp, gq, gk):
    z_btc = jnp.transpose(z, (0, 2, 1))
    zhat, perp = pl.pallas_call(
        _fused_kernel,
        grid=(2 * NV,),
        in_specs=[
            pl.BlockSpec((B, T, C), lambda i: (0, 0, 0)),
            pl.BlockSpec((VB, C), lambda i: (i % NV, 0)),
            pl.BlockSpec((C, C), lambda i: (0, 0)),
            pl.BlockSpec((1, C), lambda i: (0, 0)),
            pl.BlockSpec((C, C), lambda i: (0, 0)),
            pl.BlockSpec((1, C), lambda i: (0, 0)),
            pl.BlockSpec((C, C), lambda i: (0, 0)),
            pl.BlockSpec((1, C), lambda i: (0, 0)),
            pl.BlockSpec((C, H), lambda i: (0, 0)),
            pl.BlockSpec((1, H), lambda i: (0, 0)),
            pl.BlockSpec((C, 1), lambda i: (0, 0)),
            pl.BlockSpec((1, C), lambda i: (0, 0)),
        ],
        out_specs=[
            pl.BlockSpec((B, C, T), lambda i: (0, 0, 0)),
            pl.BlockSpec((1, 1), lambda i: (0, 0)),
        ],
        out_shape=[
            jax.ShapeDtypeStruct((B, C, T), jnp.float32),
            jax.ShapeDtypeStruct((1, 1), jnp.float32),
        ],
        scratch_shapes=[
            pltpu.VMEM((C, P), jnp.float32),
            pltpu.VMEM((1, P), jnp.float32),
            pltpu.VMEM((1, P), jnp.int32),
            pltpu.VMEM((P, 1), jnp.int32),
            pltpu.VMEM((C, P_POOL), jnp.float32),
            pltpu.VMEM((1, 1), jnp.float32),
        ],
    )(z_btc, codebook, Wq.T, bq.reshape(1, C), Wk.T, bk.reshape(1, C), Wv.T,
      bv.reshape(1, C), Wp.T, bp.reshape(1, H),
      jnp.tile(gq, H).reshape(C, 1), jnp.tile(gk, H).reshape(1, C))
    return zhat, perp[0, 0]


def kernel(z, q, codebook, Wq, bq, Wk, bk, Wv, bv, Wp, bp, gq, gk):
    del q
    return _run(z, codebook, Wq, bq, Wk, bk, Wv, bv, Wp, bp, gq, gk)


# speedup vs baseline: 2.1098x; 1.5090x over previous
import functools
import math

import jax
import jax.numpy as jnp
from jax import lax
from jax.experimental import pallas as pl
from jax.experimental.pallas import tpu as pltpu

B, C, T, V, Q, H = 4, 64, 1024, 8192, 256, 8
DH = C // H
VB = 1024
NV = V // VB
P_POOL = B * Q
P_FULL = B * T
P = P_POOL + P_FULL
EPS = 1e-5
FBIG = 1e9


def _rms_rows(x, g_tile):
    pieces = []
    for h in range(H):
        xh = x[:, h * DH:(h + 1) * DH]
        ss = jnp.sum(xh * xh, axis=1, keepdims=True)
        inv = lax.rsqrt(ss * (1.0 / DH) + EPS)
        pieces.append(jnp.broadcast_to(inv, xh.shape))
    return x * jnp.concatenate(pieces, axis=1) * g_tile


def _rms_cols(x, g_col):
    pieces = []
    for h in range(H):
        xh = x[h * DH:(h + 1) * DH, :]
        ss = jnp.sum(xh * xh, axis=0, keepdims=True)
        inv = lax.rsqrt(ss * (1.0 / DH) + EPS)
        pieces.append(jnp.broadcast_to(inv, xh.shape))
    return x * jnp.concatenate(pieces, axis=0) * g_col


def _expand_head_rows(c):
    return jnp.concatenate(
        [jnp.broadcast_to(c[h:h + 1, :], (DH, c.shape[1])) for h in range(H)],
        axis=0)


def _interp_matrix():
    t = lax.broadcasted_iota(jnp.int32, (Q, T), 1).astype(jnp.float32)
    s = lax.broadcasted_iota(jnp.int32, (Q, T), 0).astype(jnp.float32)
    src = jnp.clip((t + 0.5) * (Q / T) - 0.5, 0.0, Q - 1.0)
    i0 = jnp.floor(src)
    w = src - i0
    i1 = jnp.minimum(i0 + 1.0, Q - 1.0)
    return (s == i0) * (1.0 - w) + (s == i1) * w


def _fused_kernel(z_btc, cb_blk, WqT, bq, WkT, bk, WvT, bv, WpT, bp, gq_col,
                  gk_col, zhat_out, perp_out, wqT_s, best_s, bidx_s, zqT_s,
                  plog_s):
    i = pl.program_id(0)

    @pl.when(i == 0)
    def _prep_queries():
        z4 = z_btc[...].reshape(B, Q, T // Q, C)
        pooled = ((z4[:, :, 0, :] + z4[:, :, 1, :]) + z4[:, :, 2, :]
                  + z4[:, :, 3, :]) * (Q / T)
        hs = jnp.concatenate(
            [pooled.reshape(P_POOL, C), z_btc[...].reshape(P_FULL, C)], axis=0)
        qv = jnp.dot(hs, WqT[...], preferred_element_type=jnp.float32) + bq[...]
        c = jnp.dot(hs, WpT[...], preferred_element_type=jnp.float32) + bp[...]
        qvrT = _rms_cols(jnp.transpose(qv, (1, 0)), gq_col[...])
        wqT_s[...] = _expand_head_rows(jnp.transpose(c, (1, 0))) * qvrT

    @pl.when(i < NV)
    def _logits_phase():
        kk = jnp.dot(cb_blk[...], WkT[...],
                     preferred_element_type=jnp.float32) + bk[...]
        kvfT = _rms_cols(jnp.transpose(kk, (1, 0)), gk_col[...])
        logits = lax.dot_general(kvfT, wqT_s[...], (((0,), (0,)), ((), ())),
                                 preferred_element_type=jnp.float32)
        tmax = jnp.max(logits, axis=0, keepdims=True)
        riota = lax.broadcasted_iota(jnp.int32, (VB, P), 0).astype(jnp.float32)
        cand = jnp.where(logits == jnp.broadcast_to(tmax, (VB, P)), riota,
                         FBIG)
        targ = jnp.min(cand, axis=0, keepdims=True) + (i * VB).astype(
            jnp.float32)

        @pl.when(i == 0)
        def _():
            best_s[...] = tmax
            bidx_s[...] = targ

        @pl.when(i > 0)
        def _():
            upd = tmax > best_s[...]
            best_s[...] = jnp.where(upd, tmax, best_s[...])
            bidx_s[...] = jnp.where(upd, targ, bidx_s[...])

    @pl.when(i >= NV)
    def _assign_phase():
        j = i - NV
        value = jnp.dot(cb_blk[...], WvT[...],
                        preferred_element_type=jnp.float32) + bv[...]
        valueT = jnp.transpose(value, (1, 0))
        riota = lax.broadcasted_iota(jnp.int32, (VB, P), 0).astype(
            jnp.float32) + (j * VB).astype(jnp.float32)
        onehotT = (jnp.broadcast_to(bidx_s[0:1, 0:P_POOL], (VB, P_POOL))
                   == riota[:, 0:P_POOL]).astype(jnp.float32)
        zq_part = lax.dot_general(valueT, onehotT, (((1,), (0,)), ((), ())),
                                  preferred_element_type=jnp.float32)
        cmpT = (jnp.broadcast_to(bidx_s[0:1, P_POOL:P], (VB, P_FULL))
                == riota[:, 0:P_FULL]).astype(jnp.float32)
        ones = jnp.full((P_FULL, 1), 1.0, jnp.float32)
        counts = lax.dot_general(cmpT, ones, (((1,), (0,)), ((), ())),
                                 preferred_element_type=jnp.float32)
        p = counts * (1.0 / P_FULL)
        part = jnp.sum(p * jnp.log(p + 1e-7), axis=(0, 1), keepdims=True)

        @pl.when(j == 0)
        def _():
            zqT_s[...] = zq_part
            plog_s[...] = part

        @pl.when(j > 0)
        def _():
            zqT_s[...] = zqT_s[...] + zq_part
            plog_s[...] = plog_s[...] + part

        @pl.when(j == NV - 1)
        def _finalize():
            M = _interp_matrix()
            for b in range(B):
                zhat_out[b, :, :] = jnp.dot(zqT_s[:, b * Q:(b + 1) * Q], M,
                                            preferred_element_type=jnp.float32)
            perp_out[...] = jnp.exp(-plog_s[...])


@functools.partial(jax.jit, static_argnames=())
def _run(z, codebook, Wq, bq, Wk, bk, Wv, bv, Wp, bp, gq, gk):
    z_btc = jnp.transpose(z, (0, 2, 1))
    zhat, perp = pl.pallas_call(
        _fused_kernel,
        grid=(2 * NV,),
        in_specs=[
            pl.BlockSpec((B, T, C), lambda i: (0, 0, 0)),
            pl.BlockSpec((VB, C), lambda i: (i % NV, 0)),
            pl.BlockSpec((C, C), lambda i: (0, 0)),
            pl.BlockSpec((1, C), lambda i: (0, 0)),
            pl.BlockSpec((C, C), lambda i: (0, 0)),
            pl.BlockSpec((1, C), lambda i: (0, 0)),
            pl.BlockSpec((C, C), lambda i: (0, 0)),
            pl.BlockSpec((1, C), lambda i: (0, 0)),
            pl.BlockSpec((C, H), lambda i: (0, 0)),
            pl.BlockSpec((1, H), lambda i: (0, 0)),
            pl.BlockSpec((C, 1), lambda i: (0, 0)),
            pl.BlockSpec((C, 1), lambda i: (0, 0)),
        ],
        out_specs=[
            pl.BlockSpec((B, C, T), lambda i: (0, 0, 0)),
            pl.BlockSpec((1, 1), lambda i: (0, 0)),
        ],
        out_shape=[
            jax.ShapeDtypeStruct((B, C, T), jnp.float32),
            jax.ShapeDtypeStruct((1, 1), jnp.float32),
        ],
        scratch_shapes=[
            pltpu.VMEM((C, P), jnp.float32),
            pltpu.VMEM((1, P), jnp.float32),
            pltpu.VMEM((1, P), jnp.float32),
            pltpu.VMEM((C, P_POOL), jnp.float32),
            pltpu.VMEM((1, 1), jnp.float32),
        ],
    )(z_btc, codebook, Wq.T, bq.reshape(1, C), Wk.T, bk.reshape(1, C), Wv.T,
      bv.reshape(1, C), Wp.T, bp.reshape(1, H),
      jnp.tile(gq, H).reshape(C, 1), jnp.tile(gk, H).reshape(C, 1))
    return zhat, perp[0, 0]


def kernel(z, q, codebook, Wq, bq, Wk, bk, Wv, bv, Wp, bp, gq, gk):
    del q
    return _run(z, codebook, Wq, bq, Wk, bk, Wv, bv, Wp, bp, gq, gk)
